# baseline (speedup 1.0000x reference)
import functools

import jax
import jax.numpy as jnp
from jax import lax
from jax.experimental import pallas as pl
from jax.experimental.pallas import tpu as pltpu
from jax.experimental.pallas import tpu_sc as plsc

NC = 2
NS = 16
NW = NC * NS
R = 50
D = 64
BL = 128
NQ = R // 2


@functools.cache
def _build_lookup(B, V):
    nb = B // R
    assert nb == NW * BL
    mesh = plsc.VectorSubcoreMesh(core_axis_name="c", subcore_axis_name="s")

    @functools.partial(
        pl.kernel,
        mesh=mesh,
        out_type=jax.ShapeDtypeStruct((R * 8, NW, 8, BL), jnp.float32),
        compiler_params=pltpu.CompilerParams(
            use_tc_tiling_on_sc=False, needs_layout_passes=False
        ),
        scratch_types=[
            pltpu.VMEM((BL * R,), jnp.int32),
            pltpu.VMEM((R * BL,), jnp.int32),
            pltpu.VMEM((2 * BL, D), jnp.float32),
            pltpu.VMEM((2 * BL, D), jnp.float32),
            pltpu.VMEM((2 * D, BL), jnp.float32),
            pltpu.VMEM((2 * D, BL), jnp.float32),
            pltpu.SemaphoreType.DMA((2,)),
            pltpu.SemaphoreType.DMA((2,)),
        ],
    )
    def lookup(
        idx_hbm, table_hbm, out_hbm, idx_raw, idx_t, g0, g1, t0, t1, gsem, wsem
    ):
        gbufs, tbufs = (g0, g1), (t0, t1)
        wid = lax.axis_index("s") * NC + lax.axis_index("c")
        pltpu.sync_copy(idx_hbm.at[pl.ds(wid * BL * R, BL * R)], idx_raw)

        lanes = jax.lax.iota(jnp.int32, 16)

        def tr_idx(r, _):
            for b0 in range(0, BL, 16):
                iv = (lanes + b0) * R + r
                idx_t[pl.ds(r * BL + b0, 16)] = plsc.load_gather(idx_raw, [iv])
            return 0

        tr_idx(0, 0)
        tr_idx(1, 0)

        def gather(q, buf):
            return pltpu.make_async_copy(
                table_hbm.at[idx_t.at[pl.ds(q * 2 * BL, 2 * BL)]],
                gbufs[buf],
                gsem.at[buf],
            )

        def outwrites(q, buf):
            return [
                pltpu.make_async_copy(
                    tbufs[buf].at[pl.ds(h * D + dt * 8, 8)],
                    out_hbm.at[(2 * q + h) * 8 + dt, wid],
                    wsem.at[buf],
                )
                for h in range(2)
                for dt in range(8)
            ]

        cks = [(lanes + k) & 15 for k in range(16)]

        gather(0, 0).start()
        lax.fori_loop(2, R, tr_idx, 0)

        def maybe(cond, fn):
            if isinstance(cond, bool):
                if cond:
                    fn()
            else:
                pl.when(cond)(fn)

        def body(q, buf, start_g=True):
            gather(q, buf).wait()
            if start_g:
                maybe(q + 1 < NQ, lambda: gather(q + 1, 1 - buf).start())

            def _wait_prev():
                for w in outwrites(q - 2, buf):
                    w.wait()

            maybe(q >= 2, _wait_prev)

            def tr_blk(i, _):
                for h in range(2):
                    ivr = lanes + (i * 16 + h * BL)
                    for d0 in range(0, D, 16):
                        for k4 in range(0, 16, 4):
                            ivcs = [cks[k4 + m] + d0 for m in range(4)]
                            xs = [
                                plsc.load_gather(gbufs[buf], [ivr, ivc])
                                for ivc in ivcs
                            ]
                            for ivc, x in zip(ivcs, xs):
                                plsc.store_scatter(
                                    tbufs[buf], [ivc + h * D, ivr - h * BL], x
                                )
                return 0

            lax.fori_loop(0, BL // 16, tr_blk, 0)
            for w in outwrites(q, buf):
                w.start()

        def pair(i, _):
            body(2 * i, 0)
            body(2 * i + 1, 1)
            return 0

        lax.fori_loop(0, NQ // 2, pair, 0)
        body(NQ - 1, 0, start_g=False)
        for w in outwrites(NQ - 2, 1):
            w.wait()
        for w in outwrites(NQ - 1, 0):
            w.wait()

    return lookup


def kernel(indices, weight):
    B = indices.size
    V, _ = weight.shape
    idx = indices.reshape(-1).astype(jnp.int32)
    out = _build_lookup(B, V)(idx, weight)
    out = out.reshape(R, 8, NW, 8, BL).transpose(2, 4, 0, 1, 3)
    return out.reshape(*indices.shape, D)


# speedup vs baseline: 1.2313x; 1.2313x over previous
import functools

import jax
import jax.numpy as jnp
from jax import lax
from jax.experimental import pallas as pl
from jax.experimental.pallas import tpu as pltpu
from jax.experimental.pallas import tpu_sc as plsc

NC = 2
NS = 16
NW = NC * NS
R = 50
D = 64
BL = 128


@functools.cache
def _build_lookup(B, V):
    nb = B // R
    assert nb == NW * BL
    mesh = plsc.VectorSubcoreMesh(core_axis_name="c", subcore_axis_name="s")

    @functools.partial(
        pl.kernel,
        mesh=mesh,
        out_type=jax.ShapeDtypeStruct((R * 8, NW, 8, BL), jnp.float32),
        compiler_params=pltpu.CompilerParams(
            use_tc_tiling_on_sc=False, needs_layout_passes=False
        ),
        scratch_types=[
            pltpu.VMEM((BL * R,), jnp.int32),
            pltpu.VMEM((R, BL), jnp.int32),
            pltpu.VMEM((BL, D), jnp.float32),
            pltpu.VMEM((BL, D), jnp.float32),
            pltpu.VMEM((D, BL), jnp.float32),
            pltpu.VMEM((D, BL), jnp.float32),
            pltpu.SemaphoreType.DMA((2,)),
            pltpu.SemaphoreType.DMA((2,)),
        ],
    )
    def lookup(
        idx_hbm, table_hbm, out_hbm, idx_raw, idx_t, g0, g1, t0, t1, gsem, wsem
    ):
        gbufs, tbufs = (g0, g1), (t0, t1)
        wid = lax.axis_index("s") * NC + lax.axis_index("c")
        pltpu.sync_copy(idx_hbm.at[pl.ds(wid * BL * R, BL * R)], idx_raw)

        lanes = jax.lax.iota(jnp.int32, 16)

        def tr_idx(r, _):
            for b0 in range(0, BL, 16):
                iv = (lanes + b0) * R + r
                idx_t[r, pl.ds(b0, 16)] = plsc.load_gather(idx_raw, [iv])
            return 0

        tr_idx(0, 0)

        def gather(r, buf):
            return pltpu.make_async_copy(
                table_hbm.at[idx_t.at[r]], gbufs[buf], gsem.at[buf]
            )

        def outwrites(r, buf):
            return [
                pltpu.make_async_copy(
                    tbufs[buf].at[pl.ds(dt * 8, 8)],
                    out_hbm.at[r * 8 + dt, wid],
                    wsem.at[buf],
                )
                for dt in range(8)
            ]

        cks = [(lanes + k) & 15 for k in range(16)]

        gather(0, 0).start()
        lax.fori_loop(1, R, tr_idx, 0)

        def body(r, buf):
            gather(r, buf).wait()

            @pl.when(r + 1 < R)
            def _():
                gather(r + 1, 1 - buf).start()

            @pl.when(r >= 2)
            def _():
                for w in outwrites(r - 2, buf):
                    w.wait()

            def tr_blk(i, _):
                ivr = lanes + i * 16
                for d0 in range(0, D, 16):
                    for k4 in range(0, 16, 4):
                        ivcs = [cks[k4 + m] + d0 for m in range(4)]
                        xs = [
                            plsc.load_gather(gbufs[buf], [ivr, ivc])
                            for ivc in ivcs
                        ]
                        for ivc, x in zip(ivcs, xs):
                            plsc.store_scatter(tbufs[buf], [ivc, ivr], x)
                return 0

            lax.fori_loop(0, BL // 16, tr_blk, 0)
            for w in outwrites(r, buf):
                w.start()

        def pair(i, _):
            body(2 * i, 0)
            body(2 * i + 1, 1)
            return 0

        lax.fori_loop(0, R // 2, pair, 0)
        for w in outwrites(R - 2, 0):
            w.wait()
        for w in outwrites(R - 1, 1):
            w.wait()

    return lookup


def kernel(indices, weight):
    B = indices.size
    V, _ = weight.shape
    idx = indices.reshape(-1).astype(jnp.int32)
    out = _build_lookup(B, V)(idx, weight)
    out = out.reshape(R, 8, NW, 8, BL).transpose(2, 4, 0, 1, 3)
    return out.reshape(*indices.shape, D)
